# Initial kernel scaffold; baseline (speedup 1.0000x reference)
import functools

import jax
import jax.numpy as jnp
from jax import lax
from jax.experimental import pallas as pl
from jax.experimental.pallas import tpu as pltpu
from jax.experimental.pallas import tpu_sc as plsc

D = 100
DP = 128
NEG = -3e38
_SC_CORES = 2
_SC_SUBCORES = 16
_NW = _SC_CORES * _SC_SUBCORES

_pallas_call = pl.pallas_call


def _chunk(bpw):
    for c in range(128, 0, -8):
        if bpw % c == 0:
            return c
    return 8


def _sc_gather(jobs):
    mesh = plsc.VectorSubcoreMesh(core_axis_name="c", subcore_axis_name="s")
    out_types = []
    scratch = []
    plans = []
    for t, idx in jobs:
        n = idx.shape[0]
        bpw = n // _NW
        ch = _chunk(bpw)
        plans.append((bpw, ch, bpw // ch))
        out_types.append(jax.ShapeDtypeStruct((n, t.shape[1]), t.dtype))
        scratch.append(pltpu.VMEM((ch,), jnp.int32))
        scratch.append(pltpu.VMEM((ch, t.shape[1]), t.dtype))
    scratch.append(pltpu.SemaphoreType.DMA)
    nj = len(jobs)

    @functools.partial(pl.kernel, mesh=mesh, out_type=out_types,
                       scratch_types=scratch)
    def gather_kernel(*refs):
        ins = refs[:2 * nj]
        outs = refs[2 * nj:3 * nj]
        scr = refs[3 * nj:]
        sem = scr[-1]
        wid = lax.axis_index("s") * _SC_CORES + lax.axis_index("c")
        for j, (bpw, ch, nch) in enumerate(plans):
            t_hbm = ins[2 * j]
            i_hbm = ins[2 * j + 1]
            o_hbm = outs[j]
            idx_v = scr[2 * j]
            rows_v = scr[2 * j + 1]

            @pl.loop(0, nch)
            def _(ci, t_hbm=t_hbm, i_hbm=i_hbm, o_hbm=o_hbm, idx_v=idx_v,
                  rows_v=rows_v, bpw=bpw, ch=ch):
                base = wid * bpw + ci * ch
                pltpu.sync_copy(i_hbm.at[pl.ds(base, ch)], idx_v)
                pltpu.async_copy(t_hbm.at[idx_v], rows_v, sem).wait()
                pltpu.sync_copy(rows_v, o_hbm.at[pl.ds(base, ch)])

    flat = []
    for t, idx in jobs:
        flat += [t, idx]
    res = gather_kernel(*flat)
    if not isinstance(res, (list, tuple)):
        res = [res]
    return list(res)


def _pad_idx(idx, mult=256):
    n = idx.shape[0]
    m = -(-n // mult) * mult
    if m == n:
        return idx
    return jnp.pad(idx, (0, m - n))


def _tc1_a2u(EQ2g3, embu_p, W2T, b2p):
    NU = embu_p.shape[0]
    BLK = 2500

    def body(eq2_ref, eu_ref, w_ref, b_ref, o_ref):
        m = (eq2_ref[:, 0, :] + eq2_ref[:, 1, :] + eq2_ref[:, 2, :]
             + eq2_ref[:, 3, :]) * 0.25
        o_ref[...] = jnp.tanh(
            jnp.dot(m + eu_ref[...], w_ref[...],
                    preferred_element_type=jnp.float32) + b_ref[...])

    return _pallas_call(
        body,
        grid=(NU // BLK,),
        in_specs=[pl.BlockSpec((BLK, 4, DP), lambda i: (i, 0, 0)),
                  pl.BlockSpec((BLK, DP), lambda i: (i, 0)),
                  pl.BlockSpec((DP, DP), lambda i: (0, 0)),
                  pl.BlockSpec((1, DP), lambda i: (0, 0))],
        out_specs=pl.BlockSpec((BLK, DP), lambda i: (i, 0)),
        out_shape=jax.ShapeDtypeStruct((NU, DP), jnp.float32),
    )(EQ2g3, embu_p, W2T, b2p)


def _tc1b_skill_tables(embq5k3, qn5k3, embs_p, W2T, b2p, W1T, b1p):
    def body(e_ref, q_ref, s_ref, w2_ref, b2_ref, w1_ref, b1_ref, o1_ref,
             o2_ref):
        iot = lax.broadcasted_iota(jnp.int32, (1, 512), 1)
        P1 = jnp.zeros((512, DP), jnp.float32)
        P2 = jnp.zeros((512, DP), jnp.float32)
        for n in range(10):
            e = e_ref[:, n, :]
            cnt = jnp.zeros((512, 512), jnp.float32)
            for jj in range(4):
                cnt += (q_ref[:, n, jj:jj + 1] == iot).astype(jnp.float32)
            mS = jnp.dot(cnt, s_ref[...],
                         preferred_element_type=jnp.float32) * 0.25
            a2 = jnp.tanh(jnp.dot(mS + e, w2_ref[...],
                                  preferred_element_type=jnp.float32)
                          + b2_ref[...])
            P1 = P1 + a2
            P2 = P2 + e
        P1 = P1 * 0.1
        P2 = P2 * 0.1
        A1 = jnp.tanh(jnp.dot(P2 + s_ref[...], w1_ref[...],
                              preferred_element_type=jnp.float32)
                      + b1_ref[...])
        B1 = jnp.tanh(jnp.dot(P1 + A1, w1_ref[...],
                              preferred_element_type=jnp.float32)
                      + b1_ref[...])
        o1_ref[...] = A1
        o2_ref[...] = B1

    fullspec = lambda shp: pl.BlockSpec(shp, lambda: tuple(0 for _ in shp))
    return _pallas_call(
        body,
        in_specs=[fullspec((512, 10, DP)), fullspec((512, 10, 16)),
                  fullspec((512, DP)), fullspec((DP, DP)), fullspec((1, DP)),
                  fullspec((DP, DP)), fullspec((1, DP))],
        out_specs=[fullspec((512, DP)), fullspec((512, DP))],
        out_shape=[jax.ShapeDtypeStruct((512, DP), jnp.float32),
                   jax.ShapeDtypeStruct((512, DP), jnp.float32)],
    )(embq5k3, qn5k3, embs_p, W2T, b2p, W1T, b1p)


def _tc2a_positions(P, n1g, skg, Eq_pos, Eq_next, Eq2_pos, Eu_pos, E1u, EUm2,
                    A2um2, rm, mm, T3, embs_p, embr_p, W0T, b0p, W1T, b1p,
                    WlastT, blastp, WllT, bllp, WihT, bihp, WqT, bqp,
                    w1s, w2s):
    PB = 608
    NBLK = P // PB

    def body(n1_ref, sk_ref, eqp_ref, eqn_ref, eq2p_ref, eup_ref, e1u_ref,
             eum2_ref, a2um2_ref, rm_ref, mm_ref, t3_ref, es_ref, er_ref,
             w0_ref, b0_ref, w1_ref, b1_ref, wl_ref, bl_ref, wll_ref, bll_ref,
             wih_ref, bih_ref, wqr_ref, bq_ref, w1s_ref, w2s_ref,
             gx_ref, qs_ref, q_ref):
        f32 = jnp.float32
        dot = lambda a, b: jnp.dot(a, b, preferred_element_type=f32)
        iot = lax.broadcasted_iota(jnp.int32, (1, 512), 1)

        cnt = jnp.zeros((PB, 512), f32)
        for jj in range(4):
            cnt += (n1_ref[:, jj:jj + 1] == iot).astype(f32)
        cm = dot(cnt, t3_ref[...]) * 0.25
        mE1 = cm[:, 0:DP]
        mA1 = cm[:, DP:2 * DP]
        mB1 = cm[:, 2 * DP:3 * DP]
        eqpos = eqp_ref[...]
        a0 = jnp.tanh(dot(mE1 + eqpos, w0_ref[...]) + b0_ref[...])
        b0q = jnp.tanh(dot(mA1 + a0, w0_ref[...]) + b0_ref[...])
        cq = jnp.tanh(dot(mB1 + b0q, w0_ref[...]) + b0_ref[...])
        aggq = jnp.tanh(dot(cq, wl_ref[...]) + bl_ref[...])

        e1u = [e1u_ref[j] for j in range(4)]
        meanEU = [(eum2_ref[4 * j + 0] + eum2_ref[4 * j + 1]
                   + eum2_ref[4 * j + 2] + eum2_ref[4 * j + 3]) * 0.25
                  for j in range(4)]
        X = jnp.concatenate([meanEU[j] + e1u[j] for j in range(4)], axis=0)
        A1U = jnp.tanh(dot(X, w1_ref[...]) + b1_ref[...])
        a1u = [A1U[j * PB:(j + 1) * PB] for j in range(4)]
        meanA2 = [(a2um2_ref[4 * j + 0] + a2um2_ref[4 * j + 1]
                   + a2um2_ref[4 * j + 2] + a2um2_ref[4 * j + 3]) * 0.25
                  for j in range(4)]
        Xb = jnp.concatenate([meanA2[j] + a1u[j] for j in range(4)], axis=0)
        B1U = jnp.tanh(dot(Xb, w1_ref[...]) + b1_ref[...])
        b1u = [B1U[j * PB:(j + 1) * PB] for j in range(4)]
        a0u = jnp.tanh(dot((e1u[0] + e1u[1] + e1u[2] + e1u[3]) * 0.25
                           + eup_ref[...], w0_ref[...]) + b0_ref[...])
        b0u = jnp.tanh(dot((a1u[0] + a1u[1] + a1u[2] + a1u[3]) * 0.25
                           + a0u, w0_ref[...]) + b0_ref[...])
        c0u = jnp.tanh(dot((b1u[0] + b1u[1] + b1u[2] + b1u[3]) * 0.25
                           + b0u, w0_ref[...]) + b0_ref[...])
        aggu = jnp.tanh(dot(c0u, wl_ref[...]) + bl_ref[...])

        mmv = mm_ref[...]
        eq = jnp.where(mmv > 0.5, aggq, eqpos)
        eq2 = jnp.where(mmv > 0.5, aggu, eq2p_ref[...])
        ehat = w1s_ref[0, 0] * eq + w2s_ref[0, 0] * eq2
        er = jnp.where(rm_ref[...] > 0.5, er_ref[1:2, :], er_ref[0:1, :])
        xcat = jnp.concatenate([ehat, er], axis=1)
        x = jnp.maximum(dot(xcat, wll_ref[...]) + bll_ref[...], 0.0)
        gx_ref[...] = dot(x, wih_ref[...]) + bih_ref[...]

        eqnext = eqn_ref[...]
        qs_ref[0] = eqnext
        q_ref[0] = jnp.tanh(dot(eqnext, wqr_ref[...]) + bq_ref[...])
        for jj in range(4):
            oh = (sk_ref[:, jj:jj + 1] == iot).astype(f32)
            se = dot(oh, es_ref[...])
            qs_ref[1 + jj] = se
            q_ref[1 + jj] = jnp.tanh(dot(se, wqr_ref[...]) + bq_ref[...])

    fullw = lambda shp: pl.BlockSpec(shp, lambda i: tuple(0 for _ in shp))
    return _pallas_call(
        body,
        grid=(NBLK,),
        in_specs=[
            pl.BlockSpec((PB, 16), lambda i: (i, 0)),
            pl.BlockSpec((PB, 16), lambda i: (i, 0)),
            pl.BlockSpec((PB, DP), lambda i: (i, 0)),
            pl.BlockSpec((PB, DP), lambda i: (i, 0)),
            pl.BlockSpec((PB, DP), lambda i: (i, 0)),
            pl.BlockSpec((PB, DP), lambda i: (i, 0)),
            pl.BlockSpec((4, PB, DP), lambda i: (0, i, 0)),
            pl.BlockSpec((16, PB, DP), lambda i: (0, i, 0)),
            pl.BlockSpec((16, PB, DP), lambda i: (0, i, 0)),
            pl.BlockSpec((PB, 1), lambda i: (i, 0)),
            pl.BlockSpec((PB, 1), lambda i: (i, 0)),
            fullw((512, 3 * DP)),
            fullw((512, DP)),
            fullw((2, DP)),
            fullw((DP, DP)), fullw((1, DP)),
            fullw((DP, DP)), fullw((1, DP)),
            fullw((DP, DP)), fullw((1, DP)),
            fullw((256, 256)), fullw((1, 256)),
            fullw((256, 512)), fullw((1, 512)),
            fullw((DP, DP)), fullw((1, DP)),
            fullw((1, 1)), fullw((1, 1)),
        ],
        out_specs=[pl.BlockSpec((PB, 512), lambda i: (i, 0)),
                   pl.BlockSpec((5, PB, DP), lambda i: (0, i, 0)),
                   pl.BlockSpec((5, PB, DP), lambda i: (0, i, 0))],
        out_shape=[jax.ShapeDtypeStruct((P, 512), jnp.float32),
                   jax.ShapeDtypeStruct((5, P, DP), jnp.float32),
                   jax.ShapeDtypeStruct((5, P, DP), jnp.float32)],
    )(n1g, skg, Eq_pos, Eq_next, Eq2_pos, Eu_pos, E1u, EUm2, A2um2, rm, mm,
      T3, embs_p, embr_p, W0T, b0p, W1T, b1p, WlastT, blastp, WllT, bllp,
      WihT, bihp, WqT, bqp, w1s, w2s)


def _tc2b_lstm(B, T, gxr, WhhT, bhhp, h0p, c0p, WkT, bkp):
    def body(gx_ref, whh_ref, bhh_ref, h0_ref, c0_ref, wk_ref, bk_ref,
             H_ref, KH_ref):
        dot = lambda a, b: jnp.dot(a, b, preferred_element_type=jnp.float32)
        h = h0_ref[...]
        c = c0_ref[...]
        for t in range(T):
            g = gx_ref[:, t, :] + dot(h, whh_ref[...]) + bhh_ref[...]
            gi = g[:, 0:DP]
            gf = g[:, DP:2 * DP]
            gg = g[:, 2 * DP:3 * DP]
            go = g[:, 3 * DP:4 * DP]
            c = jax.nn.sigmoid(gf) * c + jax.nn.sigmoid(gi) * jnp.tanh(gg)
            h = jax.nn.sigmoid(go) * jnp.tanh(c)
            H_ref[:, t:t + 1, :] = h[:, None, :]
            kh = jnp.tanh(dot(h, wk_ref[...]) + bk_ref[...])
            KH_ref[:, t:t + 1, :] = kh[:, None, :]

    fullspec = lambda shp: pl.BlockSpec(shp, lambda: tuple(0 for _ in shp))
    return _pallas_call(
        body,
        in_specs=[fullspec((B, T, 4 * DP)), fullspec((DP, 4 * DP)),
                  fullspec((1, 4 * DP)), fullspec((B, DP)), fullspec((B, DP)),
                  fullspec((DP, DP)), fullspec((1, DP))],
        out_specs=[fullspec((B, T, DP)), fullspec((B, T, DP))],
        out_shape=[jax.ShapeDtypeStruct((B, T, DP), jnp.float32),
                   jax.ShapeDtypeStruct((B, T, DP), jnp.float32)],
    )(gxr, WhhT, bhhp, h0p, c0p, WkT, bkp)


def _tc2c_predict(B, T, RK, Hb, KHb, Eqb, QS, Qb, wqv, wkv, bws):
    def body(H_ref, KH_ref, eq_ref, qs_ref, qb_ref, wq_ref, wk_ref, bw_ref,
             y_ref):
        f32 = jnp.float32
        eq = eq_ref[...]
        nrm = jnp.sqrt(jnp.sum(eq * eq, axis=2))
        qn = eq / (nrm[:, :, None] + 1e-8)
        KHv = KH_ref[...]
        kwv = jnp.sum(KHv * wk_ref[...][None, :, :], axis=2)
        qwall = jnp.sum(qb_ref[...] * wq_ref[...][None, :, :], axis=2)
        Hv = H_ref[...]
        iotaL = lax.broadcasted_iota(jnp.int32, (B, T), 1)
        bw = bw_ref[0, 0]
        for t in range(T):
            qsel = qn[:, t + 1, :]
            simt = jnp.sum(qn[:, 0:T, :] * qsel[:, None, :], axis=2)
            sm = jnp.where(iotaL < t, simt, NEG)
            s = jnp.zeros((B, T), f32)
            for _ in range(min(RK, t)):
                vmax = jnp.max(sm, axis=1, keepdims=True)
                hit = sm >= vmax
                idxv = jnp.min(jnp.where(hit, iotaL, 10000), axis=1,
                               keepdims=True)
                oh = iotaL == idxv
                s = jnp.where(oh, 1.0, s)
                sm = jnp.where(oh, NEG, sm)
            num = jnp.zeros((B, 1), f32)
            Z = jnp.zeros((B, 1), f32)
            rows_w = []
            rows_v = []
            for i in range(5):
                ri = i * T + t
                qw_i = qwall[:, ri:ri + 1]
                val_i = jax.nn.sigmoid(
                    jnp.sum(Hv * qs_ref[i, t][:, None, :], axis=2))
                w0 = qw_i + kwv[:, t:t + 1] + bw
                wh = jnp.where(s > 0.5, qw_i + kwv + bw, NEG)
                rows_w.append(jnp.concatenate([w0, wh], axis=1))
                rows_v.append(jnp.concatenate([val_i[:, t:t + 1], val_i],
                                              axis=1))
            m_ = rows_w[0].max(axis=1, keepdims=True)
            for i in range(1, 5):
                m_ = jnp.maximum(m_, rows_w[i].max(axis=1, keepdims=True))
            for i in range(5):
                e_i = jnp.exp(rows_w[i] - m_)
                Z = Z + jnp.sum(e_i, axis=1, keepdims=True)
                num = num + jnp.sum(e_i * rows_v[i], axis=1, keepdims=True)
            y_ref[:, t:t + 1] = num / Z

    fullspec = lambda shp: pl.BlockSpec(shp, lambda: tuple(0 for _ in shp))
    return _pallas_call(
        body,
        in_specs=[fullspec((B, T, DP)), fullspec((B, T, DP)),
                  fullspec((B, T + 1, DP)), fullspec((5, T, B, DP)),
                  fullspec((B, 5 * T, DP)), fullspec((1, DP)),
                  fullspec((1, DP)), fullspec((1, 1))],
        out_specs=fullspec((B, T)),
        out_shape=jax.ShapeDtypeStruct((B, T), jnp.float32),
    )(Hb, KHb, Eqb, QS, Qb, wqv, wkv, bws)


def kernel(user, question, response, mask, q_neighbors, s_neighbors,
           u_neighbors, q_neighbors_2, qs_skill_ids, emb_q, emb_q2, emb_s,
           emb_u, emb_r, w1_q, w2_q, W_ll, b_ll, W_ih, W_hh, b_ih, b_hh,
           W_agg, b_agg, W_last, b_last, W_query, b_query, W_key, b_key,
           W_w, b_w, h0, c0):
    B, S = question.shape
    T = S - 1
    P = B * T
    NU = emb_u.shape[0]
    NS = emb_s.shape[0]
    RK = 10
    f32 = jnp.float32

    padc = lambda a: jnp.pad(a, ((0, 0), (0, DP - a.shape[1])))
    embq_p = padc(emb_q)
    embq2_p = padc(emb_q2)
    embu_p = padc(emb_u)
    embs_p = jnp.pad(emb_s, ((0, 512 - NS), (0, DP - D)))
    embr_p = padc(emb_r)
    padi = lambda a: jnp.pad(a, ((0, 0), (0, 16 - a.shape[1])))
    qn_t = padi(q_neighbors)
    un_t = padi(u_neighbors)
    qn2_t = padi(q_neighbors_2)
    sk_t = padi(qs_skill_ids)
    snp = jnp.pad(s_neighbors, ((0, 512 - NS), (0, 0)))

    pad_sq = lambda w: jnp.pad(w, ((0, DP - w.shape[0]), (0, DP - w.shape[1])))
    pad_b = lambda b: jnp.pad(b, (0, DP - b.shape[0])).reshape(1, DP)
    W0T = pad_sq(W_agg[0]).T
    W1T = pad_sq(W_agg[1]).T
    W2T = pad_sq(W_agg[2]).T
    b0p = pad_b(b_agg[0])
    b1p = pad_b(b_agg[1])
    b2p = pad_b(b_agg[2])
    WlastT = pad_sq(W_last).T
    blastp = pad_b(b_last)
    WqT = pad_sq(W_query).T
    bqp = pad_b(b_query)
    WkT = pad_sq(W_key).T
    bkp = pad_b(b_key)
    Wllp = jnp.pad(W_ll.reshape(2, D, 2, D),
                   ((0, 0), (0, DP - D), (0, 0), (0, DP - D))).reshape(256, 256)
    WllT = Wllp.T
    bllp = jnp.pad(b_ll.reshape(2, D), ((0, 0), (0, DP - D))).reshape(1, 256)
    Wihp = jnp.pad(W_ih.reshape(4, D, 2, D),
                   ((0, 0), (0, DP - D), (0, 0), (0, DP - D))).reshape(512, 256)
    WihT = Wihp.T
    bihp = jnp.pad(b_ih.reshape(4, D), ((0, 0), (0, DP - D))).reshape(1, 512)
    Whhp = jnp.pad(W_hh.reshape(4, D, D),
                   ((0, 0), (0, DP - D), (0, DP - D))).reshape(512, DP)
    WhhT = Whhp.T
    bhhp = jnp.pad(b_hh.reshape(4, D), ((0, 0), (0, DP - D))).reshape(1, 512)
    wqv = jnp.pad(W_w[0, :D], (0, DP - D)).reshape(1, DP)
    wkv = jnp.pad(W_w[0, D:], (0, DP - D)).reshape(1, DP)
    bws = b_w.reshape(1, 1)
    h0p = padc(h0)
    c0p = padc(c0)
    w1s = w1_q.reshape(1, 1)
    w2s = w2_q.reshape(1, 1)

    qpos = question[:, :T].reshape(-1)
    upos = user[:, :T].reshape(-1)
    qnext = question[:, 1:].reshape(-1)
    qflat = question.reshape(-1)
    snf = snp.reshape(-1)
    unf = u_neighbors.reshape(-1)
    idxA = _pad_idx(jnp.concatenate([qpos, snf]))
    idxB = jnp.concatenate([qflat, snf])
    idxC = jnp.concatenate([unf, qpos])
    idxU = _pad_idx(upos)
    idxN = _pad_idx(qnext)

    A_g, m1g, skg0, B_g, C_g, Eu_g = _sc_gather([
        (qn_t, idxA), (un_t, idxU), (sk_t, idxN),
        (embq_p, idxB), (embq2_p, idxC), (embu_p, idxU)])
    n1g = A_g[:P]
    qn5k3 = A_g[P:P + 5120].reshape(512, 10, 16)
    m1 = m1g[:P, :4]
    skg = skg0[:P]
    Eq_all = B_g[:B * S].reshape(B, S, DP)
    embq5k3 = B_g[B * S:].reshape(512, 10, DP)
    EQ2g3 = C_g[:NU * 4].reshape(NU, 4, DP)
    Eq2_pos = C_g[NU * 4:]
    Eu_pos = Eu_g[:P]

    m1jT = _pad_idx(m1.T.reshape(-1))
    m2g, E1ug = _sc_gather([(qn2_t, m1jT), (embq2_p, m1jT)])
    m2 = (m2g[:4 * P, :4].reshape(4, P, 4).transpose(0, 2, 1).reshape(-1))
    E1u = E1ug[:4 * P].reshape(4, P, DP)

    A2u_p = _tc1_a2u(EQ2g3, embu_p, W2T, b2p)

    (EUm2g,) = _sc_gather([(embu_p, m2)])
    (A2um2g,) = _sc_gather([(A2u_p, m2)])
    EUm2 = EUm2g.reshape(16, P, DP)
    A2um2 = A2um2g.reshape(16, P, DP)

    A1tab, B1tab = _tc1b_skill_tables(embq5k3, qn5k3, embs_p, W2T, b2p,
                                      W1T, b1p)
    T3 = jnp.concatenate([embs_p, A1tab, B1tab], axis=1)

    Eq_pos = Eq_all[:, :T].reshape(P, DP)
    Eq_next = Eq_all[:, 1:].reshape(P, DP)
    rm = (response[:, :T].reshape(P, 1) == 1).astype(f32)
    mm = (mask[:, :T].reshape(P, 1) == 1).astype(f32)
    gx, qs5, Q5 = _tc2a_positions(
        P, n1g, skg, Eq_pos, Eq_next, Eq2_pos, Eu_pos, E1u, EUm2, A2um2,
        rm, mm, T3, embs_p, embr_p, W0T, b0p, W1T, b1p, WlastT, blastp,
        WllT, bllp, WihT, bihp, WqT, bqp, w1s, w2s)

    gxr = gx.reshape(B, T, 4 * DP)
    Hb, KHb = _tc2b_lstm(B, T, gxr, WhhT, bhhp, h0p, c0p, WkT, bkp)

    QS = qs5.reshape(5, B, T, DP).transpose(0, 2, 1, 3)
    Qb = Q5.reshape(5, B, T, DP).transpose(1, 0, 2, 3).reshape(B, 5 * T, DP)
    y19 = _tc2c_predict(B, T, RK, Hb, KHb, Eq_all, QS, Qb, wqv, wkv, bws)
    return jnp.concatenate([jnp.full((B, 1), 0.5, f32), y19], axis=1)


# speedup vs baseline: 4.7950x; 4.7950x over previous
import functools

import jax
import jax.numpy as jnp
from jax import lax
from jax.experimental import pallas as pl
from jax.experimental.pallas import tpu as pltpu
from jax.experimental.pallas import tpu_sc as plsc

D = 100
DP = 128
NEG = -3e38
_SC_CORES = 2
_SC_SUBCORES = 16
_NW = _SC_CORES * _SC_SUBCORES

_pallas_call = pl.pallas_call


def _chunk(bpw):
    for c in range(128, 0, -8):
        if bpw % c == 0:
            return c
    return 8


def _sc_gather(jobs):
    mesh = plsc.VectorSubcoreMesh(core_axis_name="c", subcore_axis_name="s")
    out_types = []
    scratch = []
    plans = []
    for t, idx in jobs:
        n = idx.shape[0]
        bpw = n // _NW
        ch = _chunk(bpw)
        plans.append((bpw, ch, bpw // ch))
        out_types.append(jax.ShapeDtypeStruct((n, t.shape[1]), t.dtype))
        scratch.append(pltpu.VMEM((ch,), jnp.int32))
        scratch.append(pltpu.VMEM((ch, t.shape[1]), t.dtype))
    scratch.append(pltpu.SemaphoreType.DMA)
    nj = len(jobs)

    @functools.partial(pl.kernel, mesh=mesh, out_type=out_types,
                       scratch_types=scratch)
    def gather_kernel(*refs):
        ins = refs[:2 * nj]
        outs = refs[2 * nj:3 * nj]
        scr = refs[3 * nj:]
        sem = scr[-1]
        wid = lax.axis_index("s") * _SC_CORES + lax.axis_index("c")
        for j, (bpw, ch, nch) in enumerate(plans):
            t_hbm = ins[2 * j]
            i_hbm = ins[2 * j + 1]
            o_hbm = outs[j]
            idx_v = scr[2 * j]
            rows_v = scr[2 * j + 1]

            @pl.loop(0, nch)
            def _(ci, t_hbm=t_hbm, i_hbm=i_hbm, o_hbm=o_hbm, idx_v=idx_v,
                  rows_v=rows_v, bpw=bpw, ch=ch):
                base = wid * bpw + ci * ch
                pltpu.sync_copy(i_hbm.at[pl.ds(base, ch)], idx_v)
                pltpu.async_copy(t_hbm.at[idx_v], rows_v, sem).wait()
                pltpu.sync_copy(rows_v, o_hbm.at[pl.ds(base, ch)])

    flat = []
    for t, idx in jobs:
        flat += [t, idx]
    res = gather_kernel(*flat)
    if not isinstance(res, (list, tuple)):
        res = [res]
    return list(res)


def _pad_idx(idx, mult=256):
    n = idx.shape[0]
    m = -(-n // mult) * mult
    if m == n:
        return idx
    return jnp.pad(idx, (0, m - n))


def _tc1_a2u(EQ2g3, embu_p, W2T, b2p):
    NU = embu_p.shape[0]
    BLK = 2000

    def body(eq2_ref, eu_ref, w_ref, b_ref, o_ref):
        m = (eq2_ref[:, 0, :] + eq2_ref[:, 1, :] + eq2_ref[:, 2, :]
             + eq2_ref[:, 3, :]) * 0.25
        o_ref[...] = jnp.tanh(
            jnp.dot(m + eu_ref[...], w_ref[...],
                    preferred_element_type=jnp.float32) + b_ref[...])

    return _pallas_call(
        body,
        grid=(NU // BLK,),
        in_specs=[pl.BlockSpec((BLK, 4, DP), lambda i: (i, 0, 0)),
                  pl.BlockSpec((BLK, DP), lambda i: (i, 0)),
                  pl.BlockSpec((DP, DP), lambda i: (0, 0)),
                  pl.BlockSpec((1, DP), lambda i: (0, 0))],
        out_specs=pl.BlockSpec((BLK, DP), lambda i: (i, 0)),
        out_shape=jax.ShapeDtypeStruct((NU, DP), jnp.float32),
    )(EQ2g3, embu_p, W2T, b2p)


def _tc1b_skill_tables(embq5kN, qn5kN, embs_p, W2T, b2p, W1T, b1p):
    def step(e_ref, q_ref, s_ref, w2_ref, b2_ref, p1_ref, p2_ref):
        n = pl.program_id(0)
        iot = lax.broadcasted_iota(jnp.int32, (1, 512), 1)
        e = e_ref[0]
        q = q_ref[0]
        cnt = jnp.zeros((512, 512), jnp.float32)
        for jj in range(4):
            cnt += (q[:, jj:jj + 1] == iot).astype(jnp.float32)
        mS = jnp.dot(cnt, s_ref[...],
                     preferred_element_type=jnp.float32) * 0.25
        a2 = jnp.tanh(jnp.dot(mS + e, w2_ref[...],
                              preferred_element_type=jnp.float32)
                      + b2_ref[...])

        @pl.when(n == 0)
        def _():
            p1_ref[...] = jnp.zeros((512, DP), jnp.float32)
            p2_ref[...] = jnp.zeros((512, DP), jnp.float32)

        p1_ref[...] += a2
        p2_ref[...] += e

    P1, P2 = _pallas_call(
        step,
        grid=(10,),
        in_specs=[pl.BlockSpec((1, 512, DP), lambda n: (n, 0, 0)),
                  pl.BlockSpec((1, 512, 16), lambda n: (n, 0, 0)),
                  pl.BlockSpec((512, DP), lambda n: (0, 0)),
                  pl.BlockSpec((DP, DP), lambda n: (0, 0)),
                  pl.BlockSpec((1, DP), lambda n: (0, 0))],
        out_specs=[pl.BlockSpec((512, DP), lambda n: (0, 0)),
                   pl.BlockSpec((512, DP), lambda n: (0, 0))],
        out_shape=[jax.ShapeDtypeStruct((512, DP), jnp.float32),
                   jax.ShapeDtypeStruct((512, DP), jnp.float32)],
    )(embq5kN, qn5kN, embs_p, W2T, b2p)

    def fin(p1_ref, p2_ref, s_ref, w1_ref, b1_ref, o1_ref, o2_ref):
        A1 = jnp.tanh(jnp.dot(p2_ref[...] * 0.1 + s_ref[...], w1_ref[...],
                              preferred_element_type=jnp.float32)
                      + b1_ref[...])
        B1 = jnp.tanh(jnp.dot(p1_ref[...] * 0.1 + A1, w1_ref[...],
                              preferred_element_type=jnp.float32)
                      + b1_ref[...])
        o1_ref[...] = A1
        o2_ref[...] = B1

    fullspec = lambda shp: pl.BlockSpec(shp, lambda: tuple(0 for _ in shp))
    return _pallas_call(
        fin,
        in_specs=[fullspec((512, DP)), fullspec((512, DP)),
                  fullspec((512, DP)), fullspec((DP, DP)), fullspec((1, DP))],
        out_specs=[fullspec((512, DP)), fullspec((512, DP))],
        out_shape=[jax.ShapeDtypeStruct((512, DP), jnp.float32),
                   jax.ShapeDtypeStruct((512, DP), jnp.float32)],
    )(P1, P2, embs_p, W1T, b1p)


def _tc2a_positions(P, n1g, skg, Eq_pos, Eq_next, Eq2_pos, Eu_pos, E1u, EUm2,
                    A2um2, rm, mm, T3, embs_p, embr_p, W0T, b0p, W1T, b1p,
                    WlastT, blastp, WllT, bllp, WihT, bihp, WqT, bqp,
                    w1s, w2s):
    PB = 608
    NBLK = P // PB

    def body(n1_ref, sk_ref, eqp_ref, eqn_ref, eq2p_ref, eup_ref, e1u_ref,
             eum2_ref, a2um2_ref, rm_ref, mm_ref, t3_ref, es_ref, er_ref,
             w0_ref, b0_ref, w1_ref, b1_ref, wl_ref, bl_ref, wll_ref, bll_ref,
             wih_ref, bih_ref, wqr_ref, bq_ref, w1s_ref, w2s_ref,
             gx_ref, qs_ref, q_ref):
        f32 = jnp.float32
        dot = lambda a, b: jnp.dot(a, b, preferred_element_type=f32)
        iot = lax.broadcasted_iota(jnp.int32, (1, 512), 1)

        cnt = jnp.zeros((PB, 512), f32)
        for jj in range(4):
            cnt += (n1_ref[:, jj:jj + 1] == iot).astype(f32)
        cm = dot(cnt, t3_ref[...]) * 0.25
        mE1 = cm[:, 0:DP]
        mA1 = cm[:, DP:2 * DP]
        mB1 = cm[:, 2 * DP:3 * DP]
        eqpos = eqp_ref[...]
        a0 = jnp.tanh(dot(mE1 + eqpos, w0_ref[...]) + b0_ref[...])
        b0q = jnp.tanh(dot(mA1 + a0, w0_ref[...]) + b0_ref[...])
        cq = jnp.tanh(dot(mB1 + b0q, w0_ref[...]) + b0_ref[...])
        aggq = jnp.tanh(dot(cq, wl_ref[...]) + bl_ref[...])

        e1u = [e1u_ref[j] for j in range(4)]
        meanEU = [(eum2_ref[4 * j + 0] + eum2_ref[4 * j + 1]
                   + eum2_ref[4 * j + 2] + eum2_ref[4 * j + 3]) * 0.25
                  for j in range(4)]
        X = jnp.concatenate([meanEU[j] + e1u[j] for j in range(4)], axis=0)
        A1U = jnp.tanh(dot(X, w1_ref[...]) + b1_ref[...])
        a1u = [A1U[j * PB:(j + 1) * PB] for j in range(4)]
        meanA2 = [(a2um2_ref[4 * j + 0] + a2um2_ref[4 * j + 1]
                   + a2um2_ref[4 * j + 2] + a2um2_ref[4 * j + 3]) * 0.25
                  for j in range(4)]
        Xb = jnp.concatenate([meanA2[j] + a1u[j] for j in range(4)], axis=0)
        B1U = jnp.tanh(dot(Xb, w1_ref[...]) + b1_ref[...])
        b1u = [B1U[j * PB:(j + 1) * PB] for j in range(4)]
        a0u = jnp.tanh(dot((e1u[0] + e1u[1] + e1u[2] + e1u[3]) * 0.25
                           + eup_ref[...], w0_ref[...]) + b0_ref[...])
        b0u = jnp.tanh(dot((a1u[0] + a1u[1] + a1u[2] + a1u[3]) * 0.25
                           + a0u, w0_ref[...]) + b0_ref[...])
        c0u = jnp.tanh(dot((b1u[0] + b1u[1] + b1u[2] + b1u[3]) * 0.25
                           + b0u, w0_ref[...]) + b0_ref[...])
        aggu = jnp.tanh(dot(c0u, wl_ref[...]) + bl_ref[...])

        mmv = mm_ref[...]
        eq = jnp.where(mmv > 0.5, aggq, eqpos)
        eq2 = jnp.where(mmv > 0.5, aggu, eq2p_ref[...])
        ehat = w1s_ref[0, 0] * eq + w2s_ref[0, 0] * eq2
        er = jnp.where(rm_ref[...] > 0.5, er_ref[1:2, :], er_ref[0:1, :])
        xcat = jnp.concatenate([ehat, er], axis=1)
        x = jnp.maximum(dot(xcat, wll_ref[...]) + bll_ref[...], 0.0)
        gx_ref[...] = dot(x, wih_ref[...]) + bih_ref[...]

        eqnext = eqn_ref[...]
        qs_ref[0] = eqnext
        q_ref[0] = jnp.tanh(dot(eqnext, wqr_ref[...]) + bq_ref[...])
        for jj in range(4):
            oh = (sk_ref[:, jj:jj + 1] == iot).astype(f32)
            se = dot(oh, es_ref[...])
            qs_ref[1 + jj] = se
            q_ref[1 + jj] = jnp.tanh(dot(se, wqr_ref[...]) + bq_ref[...])

    fullw = lambda shp: pl.BlockSpec(shp, lambda i: tuple(0 for _ in shp))
    return _pallas_call(
        body,
        grid=(NBLK,),
        in_specs=[
            pl.BlockSpec((PB, 16), lambda i: (i, 0)),
            pl.BlockSpec((PB, 16), lambda i: (i, 0)),
            pl.BlockSpec((PB, DP), lambda i: (i, 0)),
            pl.BlockSpec((PB, DP), lambda i: (i, 0)),
            pl.BlockSpec((PB, DP), lambda i: (i, 0)),
            pl.BlockSpec((PB, DP), lambda i: (i, 0)),
            pl.BlockSpec((4, PB, DP), lambda i: (0, i, 0)),
            pl.BlockSpec((16, PB, DP), lambda i: (0, i, 0)),
            pl.BlockSpec((16, PB, DP), lambda i: (0, i, 0)),
            pl.BlockSpec((PB, 1), lambda i: (i, 0)),
            pl.BlockSpec((PB, 1), lambda i: (i, 0)),
            fullw((512, 3 * DP)),
            fullw((512, DP)),
            fullw((2, DP)),
            fullw((DP, DP)), fullw((1, DP)),
            fullw((DP, DP)), fullw((1, DP)),
            fullw((DP, DP)), fullw((1, DP)),
            fullw((256, 256)), fullw((1, 256)),
            fullw((256, 512)), fullw((1, 512)),
            fullw((DP, DP)), fullw((1, DP)),
            fullw((1, 1)), fullw((1, 1)),
        ],
        out_specs=[pl.BlockSpec((PB, 512), lambda i: (i, 0)),
                   pl.BlockSpec((5, PB, DP), lambda i: (0, i, 0)),
                   pl.BlockSpec((5, PB, DP), lambda i: (0, i, 0))],
        out_shape=[jax.ShapeDtypeStruct((P, 512), jnp.float32),
                   jax.ShapeDtypeStruct((5, P, DP), jnp.float32),
                   jax.ShapeDtypeStruct((5, P, DP), jnp.float32)],
    )(n1g, skg, Eq_pos, Eq_next, Eq2_pos, Eu_pos, E1u, EUm2, A2um2, rm, mm,
      T3, embs_p, embr_p, W0T, b0p, W1T, b1p, WlastT, blastp, WllT, bllp,
      WihT, bihp, WqT, bqp, w1s, w2s)


def _tc2b_lstm(B, T, gxr, WhhT, bhhp, h0p, c0p, WkT, bkp):
    def body(gx_ref, whh_ref, bhh_ref, h0_ref, c0_ref, wk_ref, bk_ref,
             H_ref, KH_ref):
        dot = lambda a, b: jnp.dot(a, b, preferred_element_type=jnp.float32)
        h = h0_ref[...]
        c = c0_ref[...]
        for t in range(T):
            g = gx_ref[:, t, :] + dot(h, whh_ref[...]) + bhh_ref[...]
            gi = g[:, 0:DP]
            gf = g[:, DP:2 * DP]
            gg = g[:, 2 * DP:3 * DP]
            go = g[:, 3 * DP:4 * DP]
            c = jax.nn.sigmoid(gf) * c + jax.nn.sigmoid(gi) * jnp.tanh(gg)
            h = jax.nn.sigmoid(go) * jnp.tanh(c)
            H_ref[:, t:t + 1, :] = h[:, None, :]
            kh = jnp.tanh(dot(h, wk_ref[...]) + bk_ref[...])
            KH_ref[:, t:t + 1, :] = kh[:, None, :]

    fullspec = lambda shp: pl.BlockSpec(shp, lambda: tuple(0 for _ in shp))
    return _pallas_call(
        body,
        in_specs=[fullspec((B, T, 4 * DP)), fullspec((DP, 4 * DP)),
                  fullspec((1, 4 * DP)), fullspec((B, DP)), fullspec((B, DP)),
                  fullspec((DP, DP)), fullspec((1, DP))],
        out_specs=[fullspec((B, T, DP)), fullspec((B, T, DP))],
        out_shape=[jax.ShapeDtypeStruct((B, T, DP), jnp.float32),
                   jax.ShapeDtypeStruct((B, T, DP), jnp.float32)],
    )(gxr, WhhT, bhhp, h0p, c0p, WkT, bkp)


def _tc2c_predict(B, T, RK, Hb, KHb, Eqb, QS, Qb, wqv, wkv, bws):
    def body(H_ref, KH_ref, eq_ref, qs_ref, qb_ref, wq_ref, wk_ref, bw_ref,
             y_ref):
        f32 = jnp.float32
        eq = eq_ref[...]
        nrm = jnp.sqrt(jnp.sum(eq * eq, axis=2))
        qn = eq / (nrm[:, :, None] + 1e-8)
        KHv = KH_ref[...]
        kwv = jnp.sum(KHv * wk_ref[...][None, :, :], axis=2)
        qwall = jnp.sum(qb_ref[...] * wq_ref[...][None, :, :], axis=2)
        Hv = H_ref[...]
        iotaL = lax.broadcasted_iota(jnp.int32, (B, T), 1)
        iota20 = lax.broadcasted_iota(jnp.int32, (1, T + 1), 1)
        iota5T = lax.broadcasted_iota(jnp.int32, (B, 5 * T), 1)
        bw = bw_ref[0, 0]

        def tstep(t, ycols):
            ohn = (iota20 == t + 1).astype(f32)
            qsel = jnp.sum(qn * ohn[:, :, None], axis=1)
            simt = jnp.sum(qn[:, 0:T, :] * qsel[:, None, :], axis=2)
            sm0 = jnp.where(iotaL < t, simt, NEG)

            def rstep(r, carry):
                s, sm = carry
                vmax = jnp.max(sm, axis=1, keepdims=True)
                hit = sm >= vmax
                idxv = jnp.min(jnp.where(hit, iotaL, 10000), axis=1,
                               keepdims=True)
                oh = (iotaL == idxv) & (r < t)
                s = jnp.where(oh, 1.0, s)
                sm = jnp.where(oh, NEG, sm)
                return s, sm

            s, _sm = lax.fori_loop(0, RK, rstep,
                                   (jnp.zeros((B, T), f32), sm0))
            oht = (iotaL == t).astype(f32)
            kwt = jnp.sum(kwv * oht, axis=1, keepdims=True)
            num = jnp.zeros((B, 1), f32)
            Z = jnp.zeros((B, 1), f32)
            rows_w = []
            rows_v = []
            for i in range(5):
                ohq = (iota5T == i * T + t).astype(f32)
                qw_i = jnp.sum(qwall * ohq, axis=1, keepdims=True)
                qsi = qs_ref[i, pl.ds(t, 1)][0]
                val_i = jax.nn.sigmoid(
                    jnp.sum(Hv * qsi[:, None, :], axis=2))
                v0 = jnp.sum(val_i * oht, axis=1, keepdims=True)
                w0 = qw_i + kwt + bw
                wh = jnp.where(s > 0.5, qw_i + kwv + bw, NEG)
                rows_w.append(jnp.concatenate([w0, wh], axis=1))
                rows_v.append(jnp.concatenate([v0, val_i], axis=1))
            m_ = rows_w[0].max(axis=1, keepdims=True)
            for i in range(1, 5):
                m_ = jnp.maximum(m_, rows_w[i].max(axis=1, keepdims=True))
            for i in range(5):
                e_i = jnp.exp(rows_w[i] - m_)
                Z = Z + jnp.sum(e_i, axis=1, keepdims=True)
                num = num + jnp.sum(e_i * rows_v[i], axis=1, keepdims=True)
            return jnp.where(iotaL == t, num / Z, ycols)

        y_ref[...] = lax.fori_loop(0, T, tstep,
                                   jnp.zeros((B, T), f32))

    fullspec = lambda shp: pl.BlockSpec(shp, lambda: tuple(0 for _ in shp))
    return _pallas_call(
        body,
        in_specs=[fullspec((B, T, DP)), fullspec((B, T, DP)),
                  fullspec((B, T + 1, DP)), fullspec((5, T, B, DP)),
                  fullspec((B, 5 * T, DP)), fullspec((1, DP)),
                  fullspec((1, DP)), fullspec((1, 1))],
        out_specs=fullspec((B, T)),
        out_shape=jax.ShapeDtypeStruct((B, T), jnp.float32),
    )(Hb, KHb, Eqb, QS, Qb, wqv, wkv, bws)


def kernel(user, question, response, mask, q_neighbors, s_neighbors,
           u_neighbors, q_neighbors_2, qs_skill_ids, emb_q, emb_q2, emb_s,
           emb_u, emb_r, w1_q, w2_q, W_ll, b_ll, W_ih, W_hh, b_ih, b_hh,
           W_agg, b_agg, W_last, b_last, W_query, b_query, W_key, b_key,
           W_w, b_w, h0, c0):
    B, S = question.shape
    T = S - 1
    P = B * T
    NU = emb_u.shape[0]
    NS = emb_s.shape[0]
    RK = 10
    f32 = jnp.float32

    padc = lambda a: jnp.pad(a, ((0, 0), (0, DP - a.shape[1])))
    embq_p = padc(emb_q)
    embq2_p = padc(emb_q2)
    embu_p = padc(emb_u)
    embs_p = jnp.pad(emb_s, ((0, 512 - NS), (0, DP - D)))
    embr_p = padc(emb_r)
    padi = lambda a: jnp.pad(a, ((0, 0), (0, DP - a.shape[1])))
    qn_t = padi(q_neighbors)
    un_t = padi(u_neighbors)
    qn2_t = padi(q_neighbors_2)
    sk_t = padi(qs_skill_ids)
    snp = jnp.pad(s_neighbors, ((0, 512 - NS), (0, 0)))

    pad_sq = lambda w: jnp.pad(w, ((0, DP - w.shape[0]), (0, DP - w.shape[1])))
    pad_b = lambda b: jnp.pad(b, (0, DP - b.shape[0])).reshape(1, DP)
    W0T = pad_sq(W_agg[0]).T
    W1T = pad_sq(W_agg[1]).T
    W2T = pad_sq(W_agg[2]).T
    b0p = pad_b(b_agg[0])
    b1p = pad_b(b_agg[1])
    b2p = pad_b(b_agg[2])
    WlastT = pad_sq(W_last).T
    blastp = pad_b(b_last)
    WqT = pad_sq(W_query).T
    bqp = pad_b(b_query)
    WkT = pad_sq(W_key).T
    bkp = pad_b(b_key)
    Wllp = jnp.pad(W_ll.reshape(2, D, 2, D),
                   ((0, 0), (0, DP - D), (0, 0), (0, DP - D))).reshape(256, 256)
    WllT = Wllp.T
    bllp = jnp.pad(b_ll.reshape(2, D), ((0, 0), (0, DP - D))).reshape(1, 256)
    Wihp = jnp.pad(W_ih.reshape(4, D, 2, D),
                   ((0, 0), (0, DP - D), (0, 0), (0, DP - D))).reshape(512, 256)
    WihT = Wihp.T
    bihp = jnp.pad(b_ih.reshape(4, D), ((0, 0), (0, DP - D))).reshape(1, 512)
    Whhp = jnp.pad(W_hh.reshape(4, D, D),
                   ((0, 0), (0, DP - D), (0, DP - D))).reshape(512, DP)
    WhhT = Whhp.T
    bhhp = jnp.pad(b_hh.reshape(4, D), ((0, 0), (0, DP - D))).reshape(1, 512)
    wqv = jnp.pad(W_w[0, :D], (0, DP - D)).reshape(1, DP)
    wkv = jnp.pad(W_w[0, D:], (0, DP - D)).reshape(1, DP)
    bws = b_w.reshape(1, 1)
    h0p = padc(h0)
    c0p = padc(c0)
    w1s = w1_q.reshape(1, 1)
    w2s = w2_q.reshape(1, 1)

    qpos = question[:, :T].reshape(-1)
    upos = user[:, :T].reshape(-1)
    qnext = question[:, 1:].reshape(-1)
    qflat = question.reshape(-1)
    snf = snp.reshape(-1)
    unf = u_neighbors.reshape(-1)
    idxA = _pad_idx(jnp.concatenate([qpos, snf]))
    idxB = jnp.concatenate([qflat, snf])
    idxC = jnp.concatenate([unf, qpos])
    idxU = _pad_idx(upos)
    idxN = _pad_idx(qnext)

    A_g, m1g, skg0, B_g, C_g, Eu_g = _sc_gather([
        (qn_t, idxA), (un_t, idxU), (sk_t, idxN),
        (embq_p, idxB), (embq2_p, idxC), (embu_p, idxU)])
    n1g = A_g[:P, :16]
    qn5kN = A_g[P:P + 5120, :16].reshape(512, 10, 16).transpose(1, 0, 2)
    m1 = m1g[:P, :4]
    skg = skg0[:P, :16]
    Eq_all = B_g[:B * S].reshape(B, S, DP)
    embq5kN = B_g[B * S:].reshape(512, 10, DP).transpose(1, 0, 2)
    EQ2g3 = C_g[:NU * 4].reshape(NU, 4, DP)
    Eq2_pos = C_g[NU * 4:]
    Eu_pos = Eu_g[:P]

    m1jT = _pad_idx(m1.T.reshape(-1))
    m2g, E1ug = _sc_gather([(qn2_t, m1jT), (embq2_p, m1jT)])
    m2 = (m2g[:4 * P, :4].reshape(4, P, 4).transpose(0, 2, 1).reshape(-1))
    E1u = E1ug[:4 * P].reshape(4, P, DP)

    A2u_p = _tc1_a2u(EQ2g3, embu_p, W2T, b2p)

    (EUm2g,) = _sc_gather([(embu_p, m2)])
    (A2um2g,) = _sc_gather([(A2u_p, m2)])
    EUm2 = EUm2g.reshape(16, P, DP)
    A2um2 = A2um2g.reshape(16, P, DP)

    A1tab, B1tab = _tc1b_skill_tables(embq5kN, qn5kN, embs_p, W2T, b2p,
                                      W1T, b1p)
    T3 = jnp.concatenate([embs_p, A1tab, B1tab], axis=1)

    Eq_pos = Eq_all[:, :T].reshape(P, DP)
    Eq_next = Eq_all[:, 1:].reshape(P, DP)
    rm = (response[:, :T].reshape(P, 1) == 1).astype(f32)
    mm = (mask[:, :T].reshape(P, 1) == 1).astype(f32)
    gx, qs5, Q5 = _tc2a_positions(
        P, n1g, skg, Eq_pos, Eq_next, Eq2_pos, Eu_pos, E1u, EUm2, A2um2,
        rm, mm, T3, embs_p, embr_p, W0T, b0p, W1T, b1p, WlastT, blastp,
        WllT, bllp, WihT, bihp, WqT, bqp, w1s, w2s)

    gxr = gx.reshape(B, T, 4 * DP)
    Hb, KHb = _tc2b_lstm(B, T, gxr, WhhT, bhhp, h0p, c0p, WkT, bkp)

    QS = qs5.reshape(5, B, T, DP).transpose(0, 2, 1, 3)
    Qb = Q5.reshape(5, B, T, DP).transpose(1, 0, 2, 3).reshape(B, 5 * T, DP)
    y19 = _tc2c_predict(B, T, RK, Hb, KHb, Eq_all, QS, Qb, wqv, wkv, bws)
    return jnp.concatenate([jnp.full((B, 1), 0.5, f32), y19], axis=1)
